# baseline (speedup 1.0000x reference)
import functools

import jax
import jax.numpy as jnp
from jax import lax
from jax.experimental import pallas as pl
from jax.experimental.pallas import tpu as pltpu
from jax.experimental.pallas import tpu_sc as plsc
import numpy as np

T = 2048
HIDDEN = 2048
E = 16
TOPK = 2
INTER = 1024
N_GROUP = 4
TOPK_GROUP = 2
SCALE = 2.5

NW = 32
TPW = T // NW
TBLK = 256


def _logits_body(x_ref, gw_ref, log_ref):
    log_ref[...] = jnp.dot(x_ref[...], gw_ref[...],
                           preferred_element_type=jnp.float32)


def _logits(x, gate_w):
    nblk = T // TBLK
    return pl.pallas_call(
        _logits_body,
        grid=(nblk,),
        in_specs=[
            pl.BlockSpec((TBLK, HIDDEN), lambda i: (i, 0)),
            pl.BlockSpec((HIDDEN, E), lambda i: (0, 0)),
        ],
        out_specs=pl.BlockSpec((TBLK, E), lambda i: (i, 0)),
        out_shape=jax.ShapeDtypeStruct((T, E), jnp.float32),
    )(x, gate_w)


def _shared_body(x_ref, res_ref, sgu_ref, sdn_ref, base_ref):
    xb = x_ref[...]
    gu = jnp.dot(xb.astype(jnp.bfloat16), sgu_ref[...],
                 preferred_element_type=jnp.float32)
    g = gu[:, :2 * INTER]
    u = gu[:, 2 * INTER:]
    act = (g * jax.nn.sigmoid(g) * u).astype(jnp.bfloat16)
    sh = jnp.dot(act, sdn_ref[...], preferred_element_type=jnp.float32)
    base_ref[...] = res_ref[...] + sh


def _shared(x, residual, sgu_b16, sdn_b16):
    nblk = T // TBLK
    return pl.pallas_call(
        _shared_body,
        grid=(nblk,),
        in_specs=[
            pl.BlockSpec((TBLK, HIDDEN), lambda i: (i, 0)),
            pl.BlockSpec((TBLK, HIDDEN), lambda i: (i, 0)),
            pl.BlockSpec((HIDDEN, 4 * INTER), lambda i: (0, 0)),
            pl.BlockSpec((2 * INTER, HIDDEN), lambda i: (0, 0)),
        ],
        out_specs=pl.BlockSpec((TBLK, HIDDEN), lambda i: (i, 0)),
        out_shape=jax.ShapeDtypeStruct((T, HIDDEN), jnp.float32),
    )(x, residual, sgu_b16, sdn_b16)


def _sc_gather(vec, idx):
    return vec.at[idx].get(mode="promise_in_bounds")


def _allsum(vec, lane):
    for k in (1, 2, 4, 8):
        vec = vec + _sc_gather(vec, lane ^ k)
    return vec


def _allmin(vec, lane):
    for k in (1, 2, 4, 8):
        vec = jnp.minimum(vec, _sc_gather(vec, lane ^ k))
    return vec


def _prefix(vec, lane):
    for k in (1, 2, 4, 8):
        sh = _sc_gather(vec, jnp.maximum(lane - k, 0))
        vec = vec + jnp.where(lane >= k, sh, 0)
    return vec


def _route_body(log_hbm, bias_hbm, comb_hbm, e0_hbm, e1_hbm, w0_hbm, w1_hbm,
                cnt_hbm, lbuf, cbuf, e0b, e1b, w0b, w1b, bb, cntb):
    c = lax.axis_index("c")
    s = lax.axis_index("s")
    wid = s * 2 + c
    base = wid * TPW
    pltpu.sync_copy(log_hbm.at[pl.ds(base, TPW)], lbuf)
    pltpu.sync_copy(bias_hbm, bb)
    bias = bb[...]
    lane = lax.broadcasted_iota(jnp.int32, (16,), 0)
    gidx = lane >> 2
    p_in1 = (gidx << 2) + ((lane + 1) & 3)
    p_in2 = (gidx << 2) + ((lane + 2) & 3)
    p_g = [(((gidx + k) & 3) << 2) + (lane & 3) for k in (1, 2, 3)]
    p_e = [(lane + k) & 15 for k in range(1, 16)]

    def tok(i, cnt):
        lg = lbuf[i, :]
        sgm = 1.0 / (1.0 + jnp.exp(-lg))
        sc = sgm + bias
        m = jnp.maximum(sc + _sc_gather(sc, p_in1), sc + _sc_gather(sc, p_in2))
        m = jnp.maximum(m, _sc_gather(m, p_in1))
        gscore = jnp.maximum(m, _sc_gather(m, p_in2))
        grank = jnp.zeros((16,), jnp.float32)
        for p in p_g:
            gr = _sc_gather(gscore, p)
            gi = p >> 2
            b = (gr > gscore) | ((gr == gscore) & (gi < gidx))
            grank = grank + jnp.where(b, 1.0, 0.0)
        masked = jnp.where(grank < float(TOPK_GROUP), sc, -1e30)
        erank = jnp.zeros((16,), jnp.float32)
        for p in p_e:
            mr = _sc_gather(masked, p)
            b = (mr > masked) | ((mr == masked) & (p < lane))
            erank = erank + jnp.where(b, 1.0, 0.0)
        sel = erank < float(TOPK)
        wsum = _allsum(jnp.where(sel, sgm, 0.0), lane)
        wnorm = sgm * SCALE / (wsum + 1e-20)
        comb = jnp.where(sel, wnorm, 0.0)
        cbuf[i, :] = comb
        e0v = _allmin(jnp.where(sel, lane, 16), lane)
        e1v = _allmin(jnp.where(sel & (lane != e0v), lane, 16), lane)
        w0v = _allsum(jnp.where(lane == e0v, comb, 0.0), lane)
        w1v = _allsum(jnp.where(lane == e1v, comb, 0.0), lane)
        e0b[i, :] = e0v
        e1b[i, :] = e1v
        w0b[i, :] = w0v
        w1b[i, :] = w1v
        return cnt + jnp.where(sel, 1, 0)

    cnt = lax.fori_loop(0, TPW, tok, jnp.zeros((16,), jnp.int32))
    cntb[...] = cnt
    pltpu.sync_copy(cbuf, comb_hbm.at[pl.ds(base, TPW)])
    pltpu.sync_copy(e0b, e0_hbm.at[pl.ds(base, TPW)])
    pltpu.sync_copy(e1b, e1_hbm.at[pl.ds(base, TPW)])
    pltpu.sync_copy(w0b, w0_hbm.at[pl.ds(base, TPW)])
    pltpu.sync_copy(w1b, w1_hbm.at[pl.ds(base, TPW)])
    pltpu.sync_copy(cntb, cnt_hbm.at[wid])


_ROUTE_OUT = [
    jax.ShapeDtypeStruct((T, E), jnp.float32),
    jax.ShapeDtypeStruct((T, E), jnp.int32),
    jax.ShapeDtypeStruct((T, E), jnp.int32),
    jax.ShapeDtypeStruct((T, E), jnp.float32),
    jax.ShapeDtypeStruct((T, E), jnp.float32),
    jax.ShapeDtypeStruct((NW, E), jnp.int32),
]


def _route(logits, e_bias):
    mesh = plsc.VectorSubcoreMesh(core_axis_name="c", subcore_axis_name="s")
    fn = pl.kernel(
        _route_body,
        mesh=mesh,
        out_type=_ROUTE_OUT,
        scratch_types=[
            pltpu.VMEM((TPW, E), jnp.float32),
            pltpu.VMEM((TPW, E), jnp.float32),
            pltpu.VMEM((TPW, E), jnp.int32),
            pltpu.VMEM((TPW, E), jnp.int32),
            pltpu.VMEM((TPW, E), jnp.float32),
            pltpu.VMEM((TPW, E), jnp.float32),
            pltpu.VMEM((E,), jnp.float32),
            pltpu.VMEM((E,), jnp.int32),
        ],
    )
    return fn(logits, e_bias)


def _dense_moe_body(x_ref, base_ref, comb_ref, w13_ref, w2_ref, out_ref):
    e = pl.program_id(1)

    @pl.when(e == 0)
    def _init():
        out_ref[...] = base_ref[...]

    xb = x_ref[...].astype(jnp.bfloat16)
    h = jnp.dot(xb, w13_ref[0], preferred_element_type=jnp.float32)
    g = h[:, :INTER]
    u = h[:, INTER:]
    act = (g * jax.nn.sigmoid(g) * u).astype(jnp.bfloat16)
    y = jnp.dot(act, w2_ref[0], preferred_element_type=jnp.float32)
    lane = lax.broadcasted_iota(jnp.int32, (TBLK, E), 1)
    w_col = jnp.sum(jnp.where(lane == e, comb_ref[...], 0.0), axis=1,
                    keepdims=True)
    out_ref[...] += w_col * y


def _dense_moe(x, base, comb, w13_b16, w2_b16):
    nblk = T // TBLK
    return pl.pallas_call(
        _dense_moe_body,
        grid=(nblk, E),
        in_specs=[
            pl.BlockSpec((TBLK, HIDDEN), lambda i, e: (i, 0)),
            pl.BlockSpec((TBLK, HIDDEN), lambda i, e: (i, 0)),
            pl.BlockSpec((TBLK, E), lambda i, e: (i, 0)),
            pl.BlockSpec((1, HIDDEN, 2 * INTER), lambda i, e: (e, 0, 0)),
            pl.BlockSpec((1, INTER, HIDDEN), lambda i, e: (e, 0, 0)),
        ],
        out_specs=pl.BlockSpec((TBLK, HIDDEN), lambda i, e: (i, 0)),
        out_shape=jax.ShapeDtypeStruct((T, HIDDEN), jnp.float32),
        compiler_params=pltpu.CompilerParams(
            dimension_semantics=("arbitrary", "arbitrary")),
    )(x, base, comb, w13_b16, w2_b16)


TILE = 256
TSHIFT = 8
NT = 32
NTV = NT // 16
SMAX = NT * TILE


def _dispatch_body(x_hbm, cnt_hbm, e0_hbm, e1_hbm,
                   disp_hbm, s0_hbm, s1_hbm, te_hbm, wk_hbm,
                   cb, xbuf, e0buf, e1buf, s0b, s1b, teb, sem0, sem1):
    c = lax.axis_index("c")
    s = lax.axis_index("s")
    wid = s * 2 + c
    lane = lax.broadcasted_iota(jnp.int32, (16,), 0)
    pltpu.sync_copy(cnt_hbm, cb)

    def acc(r, t):
        return t + cb[r, :]

    tot = lax.fori_loop(0, NW, acc, jnp.zeros((16,), jnp.int32))
    mycum = lax.fori_loop(0, wid, acc, jnp.zeros((16,), jnp.int32))
    pad = (tot + (TILE - 1)) & (-TILE)
    cs = _prefix(pad, lane)
    offs = cs - pad
    basev = offs + mycum

    @pl.when(wid == 0)
    def _meta():
        end_tile = cs >> TSHIFT
        ttiles = _allsum(pad, lane) >> TSHIFT
        for h in range(NTV):
            jv = lane + 16 * h
            tev = jnp.zeros((16,), jnp.int32)
            for e in range(E):
                etv = _sc_gather(end_tile, lane * 0 + e)
                tev = tev + jnp.where(jv >= etv, 1, 0)
            teb[h, :] = jnp.minimum(tev, E - 1)
            teb[NTV + h, :] = jnp.where(jv < ttiles, 1, 0)
        pltpu.sync_copy(teb.at[pl.ds(0, NTV)], te_hbm)
        pltpu.sync_copy(teb.at[pl.ds(NTV, NTV)], wk_hbm)

    def blk(b, run):
        t0 = wid * TPW + b * 16
        pltpu.sync_copy(x_hbm.at[pl.ds(t0, 16)], xbuf)
        pltpu.sync_copy(e0_hbm.at[pl.ds(t0, 16)], e0buf)
        pltpu.sync_copy(e1_hbm.at[pl.ds(t0, 16)], e1buf)
        sv0 = jnp.zeros((16,), jnp.int32)
        sv1 = jnp.zeros((16,), jnp.int32)
        for j in range(16):
            e0v = e0buf[j, :]
            s0v = _sc_gather(basev + run, e0v)
            run = run + jnp.where(lane == e0v, 1, 0)
            e1v = e1buf[j, :]
            s1v = _sc_gather(basev + run, e1v)
            run = run + jnp.where(lane == e1v, 1, 0)
            sv0 = jnp.where(lane == j, s0v, sv0)
            sv1 = jnp.where(lane == j, s1v, sv1)
        s0b[b, :] = sv0
        s1b[b, :] = sv1
        cp0 = pltpu.async_copy(xbuf, disp_hbm.at[sv0], sem0)
        cp1 = pltpu.async_copy(xbuf, disp_hbm.at[sv1], sem1)
        cp0.wait()
        cp1.wait()
        return run

    lax.fori_loop(0, TPW // 16, blk, jnp.zeros((16,), jnp.int32))
    pltpu.sync_copy(s0b, s0_hbm.at[pl.ds(wid * 4, 4)])
    pltpu.sync_copy(s1b, s1_hbm.at[pl.ds(wid * 4, 4)])


def _dispatch(x, cnt, e0, e1):
    mesh = plsc.VectorSubcoreMesh(core_axis_name="c", subcore_axis_name="s")
    fn = pl.kernel(
        _dispatch_body,
        mesh=mesh,
        out_type=[
            jax.ShapeDtypeStruct((SMAX, HIDDEN // 2), jnp.int32),
            jax.ShapeDtypeStruct((T // 16, 16), jnp.int32),
            jax.ShapeDtypeStruct((T // 16, 16), jnp.int32),
            jax.ShapeDtypeStruct((NTV, 16), jnp.int32),
            jax.ShapeDtypeStruct((NTV, 16), jnp.int32),
        ],
        scratch_types=[
            pltpu.VMEM((NW, E), jnp.int32),
            pltpu.VMEM((16, HIDDEN // 2), jnp.int32),
            pltpu.VMEM((16, E), jnp.int32),
            pltpu.VMEM((16, E), jnp.int32),
            pltpu.VMEM((4, 16), jnp.int32),
            pltpu.VMEM((4, 16), jnp.int32),
            pltpu.VMEM((2 * NTV, 16), jnp.int32),
            pltpu.SemaphoreType.DMA,
            pltpu.SemaphoreType.DMA,
        ],
    )
    return fn(x, cnt, e0, e1)


def _group_body(te_ref, wk_ref, disp_ref, w13_ref, w2_ref, y_ref, h_acc):
    i = pl.program_id(0)
    k = pl.program_id(1)
    hp = HIDDEN // 2

    @pl.when((wk_ref[i] > 0) & (k == 0))
    def _k0():
        wi = disp_ref[...]
        xe = lax.bitcast_convert_type(wi << 16, jnp.float32).astype(jnp.bfloat16)
        h_acc[...] = jnp.dot(xe, w13_ref[0].astype(jnp.bfloat16),
                             preferred_element_type=jnp.float32)

    @pl.when((wk_ref[i] > 0) & (k == 1))
    def _k1():
        wi = disp_ref[...]
        xo = lax.bitcast_convert_type(wi & (-65536), jnp.float32).astype(jnp.bfloat16)
        h = h_acc[...] + jnp.dot(xo, w13_ref[0].astype(jnp.bfloat16),
                                 preferred_element_type=jnp.float32)
        g = h[:, :INTER]
        u = h[:, INTER:]
        act = (g * jax.nn.sigmoid(g) * u).astype(jnp.bfloat16)
        yf = jnp.dot(act, w2_ref[0].astype(jnp.bfloat16),
                     preferred_element_type=jnp.float32)
        lo = ((lax.bitcast_convert_type(yf[:, :hp], jnp.int32) + 0x8000)
              >> 16) & 0xFFFF
        hi = ((lax.bitcast_convert_type(yf[:, hp:], jnp.int32) + 0x8000)
              >> 16) << 16
        y_ref[...] = hi | lo


def _grouped(disp, w13_b16, w2_b16, te, wk):
    return pl.pallas_call(
        _group_body,
        grid_spec=pltpu.PrefetchScalarGridSpec(
            num_scalar_prefetch=2,
            grid=(NT, 2),
            in_specs=[
                pl.BlockSpec((TILE, HIDDEN // 2), lambda i, k, te, wk: (i, 0)),
                pl.BlockSpec((1, HIDDEN // 2, 2 * INTER),
                             lambda i, k, te, wk: (te[i], k, 0)),
                pl.BlockSpec((1, INTER, HIDDEN),
                             lambda i, k, te, wk: (te[i], 0, 0)),
            ],
            out_specs=pl.BlockSpec((TILE, HIDDEN // 2),
                                   lambda i, k, te, wk: (i, 0)),
            scratch_shapes=[pltpu.VMEM((TILE, 2 * INTER), jnp.float32)],
        ),
        out_shape=jax.ShapeDtypeStruct((SMAX, HIDDEN // 2), jnp.int32),
        compiler_params=pltpu.CompilerParams(
            dimension_semantics=("arbitrary", "arbitrary")),
    )(te, wk, disp, w13_b16, w2_b16)


_CHT = 8
_NCH = TPW // _CHT


def _combine_body(base_hbm, y_hbm, s0_hbm, s1_hbm, w0_hbm, w1_hbm, out_hbm,
                  bbuf0, bbuf1, y0b0, y0b1, y1b0, y1b1, w0buf, w1buf,
                  sall0, sall1,
                  semb0, semb1, semg00, semg01, semg10, semg11, semo0, semo1):
    c = lax.axis_index("c")
    s = lax.axis_index("s")
    wid = s * 2 + c
    tbase = wid * TPW
    bb = (bbuf0, bbuf1)
    y0b = (y0b0, y0b1)
    y1b = (y1b0, y1b1)
    semb = (semb0, semb1)
    semg0 = (semg00, semg01)
    semg1 = (semg10, semg11)
    semo = (semo0, semo1)
    hp = HIDDEN // 2
    pltpu.sync_copy(s0_hbm.at[pl.ds(wid * 4, 4)], sall0)
    pltpu.sync_copy(s1_hbm.at[pl.ds(wid * 4, 4)], sall1)
    pltpu.sync_copy(w0_hbm.at[pl.ds(tbase, TPW)], w0buf)
    pltpu.sync_copy(w1_hbm.at[pl.ds(tbase, TPW)], w1buf)

    def issue(ch):
        sl = ch % 2
        i0 = sall0.at[ch // 2, pl.ds((ch % 2) * _CHT, _CHT)]
        i1 = sall1.at[ch // 2, pl.ds((ch % 2) * _CHT, _CHT)]
        cb = pltpu.async_copy(base_hbm.at[pl.ds(tbase + ch * _CHT, _CHT)],
                              bb[sl], semb[sl])
        g0 = pltpu.async_copy(y_hbm.at[i0], y0b[sl], semg0[sl])
        g1 = pltpu.async_copy(y_hbm.at[i1], y1b[sl], semg1[sl])
        return cb, g0, g1

    pend = issue(0)
    wr = [None, None]
    for ch in range(_NCH):
        sl = ch % 2
        for p in pend:
            p.wait()
        if ch + 1 < _NCH:
            if wr[(ch + 1) % 2] is not None:
                wr[(ch + 1) % 2].wait()
            pend = issue(ch + 1)

        def row(j, cc):
            w0v = w0buf[ch * _CHT + j, :]
            w1v = w1buf[ch * _CHT + j, :]

            def col(k, cc2):
                for kk in range(2):
                    slc = pl.ds((k * 2 + kk) * 16, 16)
                    slh = pl.ds((k * 2 + kk) * 16 + hp, 16)
                    p0 = y0b[sl][j, slc]
                    p1 = y1b[sl][j, slc]
                    lo0 = lax.bitcast_convert_type(p0 << 16, jnp.float32)
                    hi0 = lax.bitcast_convert_type(p0 & (-65536), jnp.float32)
                    lo1 = lax.bitcast_convert_type(p1 << 16, jnp.float32)
                    hi1 = lax.bitcast_convert_type(p1 & (-65536), jnp.float32)
                    bb[sl][j, slc] = (bb[sl][j, slc] + w0v * lo0 + w1v * lo1)
                    bb[sl][j, slh] = (bb[sl][j, slh] + w0v * hi0 + w1v * hi1)
                return cc2

            lax.fori_loop(0, hp // 32, col, 0)
            return cc

        lax.fori_loop(0, _CHT, row, 0)
        wr[sl] = pltpu.async_copy(
            bb[sl], out_hbm.at[pl.ds(tbase + ch * _CHT, _CHT)], semo[sl])
    for w in wr:
        if w is not None:
            w.wait()


def _combine(base, y, s0, s1, w0, w1):
    mesh = plsc.VectorSubcoreMesh(core_axis_name="c", subcore_axis_name="s")
    fn = pl.kernel(
        _combine_body,
        mesh=mesh,
        out_type=[jax.ShapeDtypeStruct((T, HIDDEN), jnp.float32)],
        scratch_types=[
            pltpu.VMEM((_CHT, HIDDEN), jnp.float32),
            pltpu.VMEM((_CHT, HIDDEN), jnp.float32),
            pltpu.VMEM((_CHT, HIDDEN // 2), jnp.int32),
            pltpu.VMEM((_CHT, HIDDEN // 2), jnp.int32),
            pltpu.VMEM((_CHT, HIDDEN // 2), jnp.int32),
            pltpu.VMEM((_CHT, HIDDEN // 2), jnp.int32),
            pltpu.VMEM((TPW, E), jnp.float32),
            pltpu.VMEM((TPW, E), jnp.float32),
            pltpu.VMEM((4, 16), jnp.int32),
            pltpu.VMEM((4, 16), jnp.int32),
            pltpu.SemaphoreType.DMA,
            pltpu.SemaphoreType.DMA,
            pltpu.SemaphoreType.DMA,
            pltpu.SemaphoreType.DMA,
            pltpu.SemaphoreType.DMA,
            pltpu.SemaphoreType.DMA,
            pltpu.SemaphoreType.DMA,
            pltpu.SemaphoreType.DMA,
        ],
    )
    return fn(base, y, s0, s1, w0, w1)


def kernel(hidden_states, residual, gate_w, e_bias, w13, w2, shared_gate_up,
           shared_down):
    sgu_b16 = shared_gate_up.astype(jnp.bfloat16)
    sdn_b16 = shared_down.astype(jnp.bfloat16)
    x_b16 = hidden_states.astype(jnp.bfloat16)
    hp = HIDDEN // 2
    xbits = lax.bitcast_convert_type(x_b16, jnp.uint16).astype(jnp.int32)
    xpk = xbits[:, :hp] | (xbits[:, hp:] << 16)
    logits = _logits(hidden_states, gate_w)
    comb, e0, e1, w0, w1, cnt = _route(logits, e_bias)
    disp, s0, s1, te2d, wk2d = _dispatch(xpk, cnt, e0, e1)
    base = _shared(hidden_states, residual, sgu_b16, sdn_b16)
    y = _grouped(disp, w13, w2, te2d.reshape(-1), wk2d.reshape(-1))
    out, = _combine(base, y, s0, s1, w0, w1)
    return out


# speedup vs baseline: 1.1233x; 1.1233x over previous
import functools

import jax
import jax.numpy as jnp
from jax import lax
from jax.experimental import pallas as pl
from jax.experimental.pallas import tpu as pltpu
from jax.experimental.pallas import tpu_sc as plsc
import numpy as np

T = 2048
HIDDEN = 2048
E = 16
TOPK = 2
INTER = 1024
N_GROUP = 4
TOPK_GROUP = 2
SCALE = 2.5

NW = 32
TPW = T // NW
TBLK = 256


def _logits_body(x_ref, gw_ref, log_ref):
    log_ref[...] = jnp.dot(x_ref[...], gw_ref[...],
                           preferred_element_type=jnp.float32)


def _logits(x, gate_w):
    nblk = T // TBLK
    return pl.pallas_call(
        _logits_body,
        grid=(nblk,),
        in_specs=[
            pl.BlockSpec((TBLK, HIDDEN), lambda i: (i, 0)),
            pl.BlockSpec((HIDDEN, E), lambda i: (0, 0)),
        ],
        out_specs=pl.BlockSpec((TBLK, E), lambda i: (i, 0)),
        out_shape=jax.ShapeDtypeStruct((T, E), jnp.float32),
    )(x, gate_w)


def _shared_body(x_ref, res_ref, sgu_ref, sdn_ref, base_ref):
    xb = x_ref[...]
    gu = jnp.dot(xb.astype(jnp.bfloat16), sgu_ref[...],
                 preferred_element_type=jnp.float32)
    g = gu[:, :2 * INTER]
    u = gu[:, 2 * INTER:]
    act = (g * jax.nn.sigmoid(g) * u).astype(jnp.bfloat16)
    sh = jnp.dot(act, sdn_ref[...], preferred_element_type=jnp.float32)
    base_ref[...] = res_ref[...] + sh


def _shared(x, residual, sgu_b16, sdn_b16):
    nblk = T // TBLK
    return pl.pallas_call(
        _shared_body,
        grid=(nblk,),
        in_specs=[
            pl.BlockSpec((TBLK, HIDDEN), lambda i: (i, 0)),
            pl.BlockSpec((TBLK, HIDDEN), lambda i: (i, 0)),
            pl.BlockSpec((HIDDEN, 4 * INTER), lambda i: (0, 0)),
            pl.BlockSpec((2 * INTER, HIDDEN), lambda i: (0, 0)),
        ],
        out_specs=pl.BlockSpec((TBLK, HIDDEN), lambda i: (i, 0)),
        out_shape=jax.ShapeDtypeStruct((T, HIDDEN), jnp.float32),
    )(x, residual, sgu_b16, sdn_b16)


def _sc_gather(vec, idx):
    return vec.at[idx].get(mode="promise_in_bounds")


def _allsum(vec, lane):
    for k in (1, 2, 4, 8):
        vec = vec + _sc_gather(vec, lane ^ k)
    return vec


def _allmin(vec, lane):
    for k in (1, 2, 4, 8):
        vec = jnp.minimum(vec, _sc_gather(vec, lane ^ k))
    return vec


def _prefix(vec, lane):
    for k in (1, 2, 4, 8):
        sh = _sc_gather(vec, jnp.maximum(lane - k, 0))
        vec = vec + jnp.where(lane >= k, sh, 0)
    return vec


def _route_body(log_hbm, bias_hbm, comb_hbm, e0_hbm, e1_hbm, w0_hbm, w1_hbm,
                cnt_hbm, lbuf, cbuf, e0b, e1b, w0b, w1b, bb, cntb):
    c = lax.axis_index("c")
    s = lax.axis_index("s")
    wid = s * 2 + c
    base = wid * TPW
    pltpu.sync_copy(log_hbm.at[pl.ds(base, TPW)], lbuf)
    pltpu.sync_copy(bias_hbm, bb)
    bias = bb[...]
    lane = lax.broadcasted_iota(jnp.int32, (16,), 0)
    gidx = lane >> 2
    p_in1 = (gidx << 2) + ((lane + 1) & 3)
    p_in2 = (gidx << 2) + ((lane + 2) & 3)
    p_g = [(((gidx + k) & 3) << 2) + (lane & 3) for k in (1, 2, 3)]
    p_e = [(lane + k) & 15 for k in range(1, 16)]

    def tok(i, cnt):
        lg = lbuf[i, :]
        sgm = 1.0 / (1.0 + jnp.exp(-lg))
        sc = sgm + bias
        m = jnp.maximum(sc + _sc_gather(sc, p_in1), sc + _sc_gather(sc, p_in2))
        m = jnp.maximum(m, _sc_gather(m, p_in1))
        gscore = jnp.maximum(m, _sc_gather(m, p_in2))
        grank = jnp.zeros((16,), jnp.float32)
        for p in p_g:
            gr = _sc_gather(gscore, p)
            gi = p >> 2
            b = (gr > gscore) | ((gr == gscore) & (gi < gidx))
            grank = grank + jnp.where(b, 1.0, 0.0)
        masked = jnp.where(grank < float(TOPK_GROUP), sc, -1e30)
        erank = jnp.zeros((16,), jnp.float32)
        for p in p_e:
            mr = _sc_gather(masked, p)
            b = (mr > masked) | ((mr == masked) & (p < lane))
            erank = erank + jnp.where(b, 1.0, 0.0)
        sel = erank < float(TOPK)
        wsum = _allsum(jnp.where(sel, sgm, 0.0), lane)
        wnorm = sgm * SCALE / (wsum + 1e-20)
        comb = jnp.where(sel, wnorm, 0.0)
        cbuf[i, :] = comb
        e0v = _allmin(jnp.where(sel, lane, 16), lane)
        e1v = _allmin(jnp.where(sel & (lane != e0v), lane, 16), lane)
        w0v = _allsum(jnp.where(lane == e0v, comb, 0.0), lane)
        w1v = _allsum(jnp.where(lane == e1v, comb, 0.0), lane)
        e0b[i, :] = e0v
        e1b[i, :] = e1v
        w0b[i, :] = w0v
        w1b[i, :] = w1v
        return cnt + jnp.where(sel, 1, 0)

    cnt = lax.fori_loop(0, TPW, tok, jnp.zeros((16,), jnp.int32))
    cntb[...] = cnt
    pltpu.sync_copy(cbuf, comb_hbm.at[pl.ds(base, TPW)])
    pltpu.sync_copy(e0b, e0_hbm.at[pl.ds(base, TPW)])
    pltpu.sync_copy(e1b, e1_hbm.at[pl.ds(base, TPW)])
    pltpu.sync_copy(w0b, w0_hbm.at[pl.ds(base, TPW)])
    pltpu.sync_copy(w1b, w1_hbm.at[pl.ds(base, TPW)])
    pltpu.sync_copy(cntb, cnt_hbm.at[wid])


_ROUTE_OUT = [
    jax.ShapeDtypeStruct((T, E), jnp.float32),
    jax.ShapeDtypeStruct((T, E), jnp.int32),
    jax.ShapeDtypeStruct((T, E), jnp.int32),
    jax.ShapeDtypeStruct((T, E), jnp.float32),
    jax.ShapeDtypeStruct((T, E), jnp.float32),
    jax.ShapeDtypeStruct((NW, E), jnp.int32),
]


def _route(logits, e_bias):
    mesh = plsc.VectorSubcoreMesh(core_axis_name="c", subcore_axis_name="s")
    fn = pl.kernel(
        _route_body,
        mesh=mesh,
        out_type=_ROUTE_OUT,
        scratch_types=[
            pltpu.VMEM((TPW, E), jnp.float32),
            pltpu.VMEM((TPW, E), jnp.float32),
            pltpu.VMEM((TPW, E), jnp.int32),
            pltpu.VMEM((TPW, E), jnp.int32),
            pltpu.VMEM((TPW, E), jnp.float32),
            pltpu.VMEM((TPW, E), jnp.float32),
            pltpu.VMEM((E,), jnp.float32),
            pltpu.VMEM((E,), jnp.int32),
        ],
    )
    return fn(logits, e_bias)


def _dense_moe_body(x_ref, base_ref, comb_ref, w13_ref, w2_ref, out_ref):
    e = pl.program_id(1)

    @pl.when(e == 0)
    def _init():
        out_ref[...] = base_ref[...]

    xb = x_ref[...].astype(jnp.bfloat16)
    h = jnp.dot(xb, w13_ref[0], preferred_element_type=jnp.float32)
    g = h[:, :INTER]
    u = h[:, INTER:]
    act = (g * jax.nn.sigmoid(g) * u).astype(jnp.bfloat16)
    y = jnp.dot(act, w2_ref[0], preferred_element_type=jnp.float32)
    lane = lax.broadcasted_iota(jnp.int32, (TBLK, E), 1)
    w_col = jnp.sum(jnp.where(lane == e, comb_ref[...], 0.0), axis=1,
                    keepdims=True)
    out_ref[...] += w_col * y


def _dense_moe(x, base, comb, w13_b16, w2_b16):
    nblk = T // TBLK
    return pl.pallas_call(
        _dense_moe_body,
        grid=(nblk, E),
        in_specs=[
            pl.BlockSpec((TBLK, HIDDEN), lambda i, e: (i, 0)),
            pl.BlockSpec((TBLK, HIDDEN), lambda i, e: (i, 0)),
            pl.BlockSpec((TBLK, E), lambda i, e: (i, 0)),
            pl.BlockSpec((1, HIDDEN, 2 * INTER), lambda i, e: (e, 0, 0)),
            pl.BlockSpec((1, INTER, HIDDEN), lambda i, e: (e, 0, 0)),
        ],
        out_specs=pl.BlockSpec((TBLK, HIDDEN), lambda i, e: (i, 0)),
        out_shape=jax.ShapeDtypeStruct((T, HIDDEN), jnp.float32),
        compiler_params=pltpu.CompilerParams(
            dimension_semantics=("arbitrary", "arbitrary")),
    )(x, base, comb, w13_b16, w2_b16)


TILE = 256
TSHIFT = 8
NT = 32
NTV = NT // 16
SMAX = NT * TILE


def _dispatch_body(x_hbm, cnt_hbm, e0_hbm, e1_hbm,
                   disp_hbm, s0_hbm, s1_hbm, te_hbm, wk_hbm,
                   cb, xbuf, e0buf, e1buf, s0b, s1b, teb, sem0, sem1):
    c = lax.axis_index("c")
    s = lax.axis_index("s")
    wid = s * 2 + c
    lane = lax.broadcasted_iota(jnp.int32, (16,), 0)
    pltpu.sync_copy(cnt_hbm, cb)

    def acc(r, t):
        return t + cb[r, :]

    tot = lax.fori_loop(0, NW, acc, jnp.zeros((16,), jnp.int32))
    mycum = lax.fori_loop(0, wid, acc, jnp.zeros((16,), jnp.int32))
    pad = (tot + (TILE - 1)) & (-TILE)
    cs = _prefix(pad, lane)
    offs = cs - pad
    basev = offs + mycum

    @pl.when(wid == 0)
    def _meta():
        end_tile = cs >> TSHIFT
        ttiles = _allsum(pad, lane) >> TSHIFT
        for h in range(NTV):
            jv = lane + 16 * h
            tev = jnp.zeros((16,), jnp.int32)
            for e in range(E):
                etv = _sc_gather(end_tile, lane * 0 + e)
                tev = tev + jnp.where(jv >= etv, 1, 0)
            teb[h, :] = jnp.minimum(tev, E - 1)
            teb[NTV + h, :] = jnp.where(jv < ttiles, 1, 0)
        pltpu.sync_copy(teb.at[pl.ds(0, NTV)], te_hbm)
        pltpu.sync_copy(teb.at[pl.ds(NTV, NTV)], wk_hbm)

    def blk(b, run):
        t0 = wid * TPW + b * 16
        pltpu.sync_copy(x_hbm.at[pl.ds(t0, 16)], xbuf)
        pltpu.sync_copy(e0_hbm.at[pl.ds(t0, 16)], e0buf)
        pltpu.sync_copy(e1_hbm.at[pl.ds(t0, 16)], e1buf)
        sv0 = jnp.zeros((16,), jnp.int32)
        sv1 = jnp.zeros((16,), jnp.int32)
        for j in range(16):
            e0v = e0buf[j, :]
            s0v = _sc_gather(basev + run, e0v)
            run = run + jnp.where(lane == e0v, 1, 0)
            e1v = e1buf[j, :]
            s1v = _sc_gather(basev + run, e1v)
            run = run + jnp.where(lane == e1v, 1, 0)
            sv0 = jnp.where(lane == j, s0v, sv0)
            sv1 = jnp.where(lane == j, s1v, sv1)
        s0b[b, :] = sv0
        s1b[b, :] = sv1
        cp0 = pltpu.async_copy(xbuf, disp_hbm.at[sv0], sem0)
        cp1 = pltpu.async_copy(xbuf, disp_hbm.at[sv1], sem1)
        cp0.wait()
        cp1.wait()
        return run

    lax.fori_loop(0, TPW // 16, blk, jnp.zeros((16,), jnp.int32))
    pltpu.sync_copy(s0b, s0_hbm.at[pl.ds(wid * 4, 4)])
    pltpu.sync_copy(s1b, s1_hbm.at[pl.ds(wid * 4, 4)])


def _dispatch(x, cnt, e0, e1):
    mesh = plsc.VectorSubcoreMesh(core_axis_name="c", subcore_axis_name="s")
    fn = pl.kernel(
        _dispatch_body,
        mesh=mesh,
        out_type=[
            jax.ShapeDtypeStruct((SMAX, HIDDEN // 2), jnp.int32),
            jax.ShapeDtypeStruct((T // 16, 16), jnp.int32),
            jax.ShapeDtypeStruct((T // 16, 16), jnp.int32),
            jax.ShapeDtypeStruct((NTV, 16), jnp.int32),
            jax.ShapeDtypeStruct((NTV, 16), jnp.int32),
        ],
        scratch_types=[
            pltpu.VMEM((NW, E), jnp.int32),
            pltpu.VMEM((16, HIDDEN // 2), jnp.int32),
            pltpu.VMEM((16, E), jnp.int32),
            pltpu.VMEM((16, E), jnp.int32),
            pltpu.VMEM((4, 16), jnp.int32),
            pltpu.VMEM((4, 16), jnp.int32),
            pltpu.VMEM((2 * NTV, 16), jnp.int32),
            pltpu.SemaphoreType.DMA,
            pltpu.SemaphoreType.DMA,
        ],
    )
    return fn(x, cnt, e0, e1)


def _group_body(te_ref, wk_ref, disp_ref, w13_ref, w2_ref, y_ref):
    i = pl.program_id(0)

    @pl.when(wk_ref[i] > 0)
    def _go():
        hp = HIDDEN // 2
        wi = disp_ref[...]
        xe = lax.bitcast_convert_type(wi << 16, jnp.float32).astype(jnp.bfloat16)
        xo = lax.bitcast_convert_type(wi & (-65536), jnp.float32).astype(jnp.bfloat16)
        we = w13_ref[0, :hp, :].astype(jnp.bfloat16)
        wo = w13_ref[0, hp:, :].astype(jnp.bfloat16)
        h = (jnp.dot(xe, we, preferred_element_type=jnp.float32)
             + jnp.dot(xo, wo, preferred_element_type=jnp.float32))
        g = h[:, :INTER]
        u = h[:, INTER:]
        act = (g * jax.nn.sigmoid(g) * u).astype(jnp.bfloat16)
        yf = jnp.dot(act, w2_ref[0].astype(jnp.bfloat16),
                     preferred_element_type=jnp.float32)
        lo = ((lax.bitcast_convert_type(yf[:, :hp], jnp.int32) + 0x8000)
              >> 16) & 0xFFFF
        hi = ((lax.bitcast_convert_type(yf[:, hp:], jnp.int32) + 0x8000)
              >> 16) << 16
        y_ref[...] = hi | lo


def _grouped(disp, w13_b16, w2_b16, te, wk):
    return pl.pallas_call(
        _group_body,
        grid_spec=pltpu.PrefetchScalarGridSpec(
            num_scalar_prefetch=2,
            grid=(NT,),
            in_specs=[
                pl.BlockSpec((TILE, HIDDEN // 2), lambda i, te, wk: (i, 0)),
                pl.BlockSpec((1, HIDDEN, 2 * INTER),
                             lambda i, te, wk: (te[i], 0, 0)),
                pl.BlockSpec((1, INTER, HIDDEN),
                             lambda i, te, wk: (te[i], 0, 0)),
            ],
            out_specs=pl.BlockSpec((TILE, HIDDEN // 2),
                                   lambda i, te, wk: (i, 0)),
        ),
        out_shape=jax.ShapeDtypeStruct((SMAX, HIDDEN // 2), jnp.int32),
        compiler_params=pltpu.CompilerParams(
            dimension_semantics=("arbitrary",)),
    )(te, wk, disp, w13_b16, w2_b16)


_CHT = 8
_NCH = TPW // _CHT


def _combine_body(base_hbm, y_hbm, s0_hbm, s1_hbm, w0_hbm, w1_hbm, out_hbm,
                  bbuf0, bbuf1, y0b0, y0b1, y1b0, y1b1, w0buf, w1buf,
                  sall0, sall1,
                  semb0, semb1, semg00, semg01, semg10, semg11, semo0, semo1):
    c = lax.axis_index("c")
    s = lax.axis_index("s")
    wid = s * 2 + c
    tbase = wid * TPW
    bb = (bbuf0, bbuf1)
    y0b = (y0b0, y0b1)
    y1b = (y1b0, y1b1)
    semb = (semb0, semb1)
    semg0 = (semg00, semg01)
    semg1 = (semg10, semg11)
    semo = (semo0, semo1)
    hp = HIDDEN // 2
    pltpu.sync_copy(s0_hbm.at[pl.ds(wid * 4, 4)], sall0)
    pltpu.sync_copy(s1_hbm.at[pl.ds(wid * 4, 4)], sall1)
    pltpu.sync_copy(w0_hbm.at[pl.ds(tbase, TPW)], w0buf)
    pltpu.sync_copy(w1_hbm.at[pl.ds(tbase, TPW)], w1buf)

    def issue(ch):
        sl = ch % 2
        i0 = sall0.at[ch // 2, pl.ds((ch % 2) * _CHT, _CHT)]
        i1 = sall1.at[ch // 2, pl.ds((ch % 2) * _CHT, _CHT)]
        cb = pltpu.async_copy(base_hbm.at[pl.ds(tbase + ch * _CHT, _CHT)],
                              bb[sl], semb[sl])
        g0 = pltpu.async_copy(y_hbm.at[i0], y0b[sl], semg0[sl])
        g1 = pltpu.async_copy(y_hbm.at[i1], y1b[sl], semg1[sl])
        return cb, g0, g1

    pend = issue(0)
    wr = [None, None]
    for ch in range(_NCH):
        sl = ch % 2
        for p in pend:
            p.wait()
        if ch + 1 < _NCH:
            if wr[(ch + 1) % 2] is not None:
                wr[(ch + 1) % 2].wait()
            pend = issue(ch + 1)

        def row(j, cc):
            w0v = w0buf[ch * _CHT + j, :]
            w1v = w1buf[ch * _CHT + j, :]

            def col(k, cc2):
                for kk in range(2):
                    slc = pl.ds((k * 2 + kk) * 16, 16)
                    slh = pl.ds((k * 2 + kk) * 16 + hp, 16)
                    p0 = y0b[sl][j, slc]
                    p1 = y1b[sl][j, slc]
                    lo0 = lax.bitcast_convert_type(p0 << 16, jnp.float32)
                    hi0 = lax.bitcast_convert_type(p0 & (-65536), jnp.float32)
                    lo1 = lax.bitcast_convert_type(p1 << 16, jnp.float32)
                    hi1 = lax.bitcast_convert_type(p1 & (-65536), jnp.float32)
                    bb[sl][j, slc] = (bb[sl][j, slc] + w0v * lo0 + w1v * lo1)
                    bb[sl][j, slh] = (bb[sl][j, slh] + w0v * hi0 + w1v * hi1)
                return cc2

            lax.fori_loop(0, hp // 32, col, 0)
            return cc

        lax.fori_loop(0, _CHT, row, 0)
        wr[sl] = pltpu.async_copy(
            bb[sl], out_hbm.at[pl.ds(tbase + ch * _CHT, _CHT)], semo[sl])
    for w in wr:
        if w is not None:
            w.wait()


def _combine(base, y, s0, s1, w0, w1):
    mesh = plsc.VectorSubcoreMesh(core_axis_name="c", subcore_axis_name="s")
    fn = pl.kernel(
        _combine_body,
        mesh=mesh,
        out_type=[jax.ShapeDtypeStruct((T, HIDDEN), jnp.float32)],
        scratch_types=[
            pltpu.VMEM((_CHT, HIDDEN), jnp.float32),
            pltpu.VMEM((_CHT, HIDDEN), jnp.float32),
            pltpu.VMEM((_CHT, HIDDEN // 2), jnp.int32),
            pltpu.VMEM((_CHT, HIDDEN // 2), jnp.int32),
            pltpu.VMEM((_CHT, HIDDEN // 2), jnp.int32),
            pltpu.VMEM((_CHT, HIDDEN // 2), jnp.int32),
            pltpu.VMEM((TPW, E), jnp.float32),
            pltpu.VMEM((TPW, E), jnp.float32),
            pltpu.VMEM((4, 16), jnp.int32),
            pltpu.VMEM((4, 16), jnp.int32),
            pltpu.SemaphoreType.DMA,
            pltpu.SemaphoreType.DMA,
            pltpu.SemaphoreType.DMA,
            pltpu.SemaphoreType.DMA,
            pltpu.SemaphoreType.DMA,
            pltpu.SemaphoreType.DMA,
            pltpu.SemaphoreType.DMA,
            pltpu.SemaphoreType.DMA,
        ],
    )
    return fn(base, y, s0, s1, w0, w1)


def kernel(hidden_states, residual, gate_w, e_bias, w13, w2, shared_gate_up,
           shared_down):
    sgu_b16 = shared_gate_up.astype(jnp.bfloat16)
    sdn_b16 = shared_down.astype(jnp.bfloat16)
    x_b16 = hidden_states.astype(jnp.bfloat16)
    hp = HIDDEN // 2
    xbits = lax.bitcast_convert_type(x_b16, jnp.uint16).astype(jnp.int32)
    xpk = xbits[:, :hp] | (xbits[:, hp:] << 16)
    logits = _logits(hidden_states, gate_w)
    comb, e0, e1, w0, w1, cnt = _route(logits, e_bias)
    disp, s0, s1, te2d, wk2d = _dispatch(xpk, cnt, e0, e1)
    base = _shared(hidden_states, residual, sgu_b16, sdn_b16)
    y = _grouped(disp, w13, w2, te2d.reshape(-1), wk2d.reshape(-1))
    out, = _combine(base, y, s0, s1, w0, w1)
    return out
